# baseline (speedup 1.0000x reference)
import jax
import jax.numpy as jnp
from jax import lax
from jax.experimental import pallas as pl
from jax.experimental.pallas import tpu as pltpu
from jax.experimental.pallas import tpu_sc as plsc

L = 16
NW = 32
B = 16384
BW = B // NW
EW = BW * 3
ICH = 128
NCH = EW // ICH


def _body(lloc_hbm, aloc_hbm, ladj_hbm, aadj_hbm, ttc_hbm, cam_hbm, sen_hbm,
          out_hbm, tout_hbm,
          camv, senv, eidx, lrow, arow, llocv, alocv, outv, outt, ttcv,
          lsem, asem):
    wid = lax.axis_index("s") * 2 + lax.axis_index("c")
    base = wid * BW

    pltpu.sync_copy(cam_hbm.at[pl.ds(base, BW)], camv)
    lane = lax.iota(jnp.int32, L)

    def expand(g, carry):
        e = camv[pl.ds(g * L, L)] * 3
        pos = (g * L + lane) * 3
        for c in range(3):
            plsc.store_scatter(eidx, [pos + c], e + c)
        return carry

    lax.fori_loop(0, BW // L, expand, 0)

    copies = []
    for j in range(NCH):
        sl = pl.ds(j * ICH, ICH)
        copies.append(pltpu.async_copy(ladj_hbm.at[eidx.at[sl]], lrow.at[sl], lsem))
        copies.append(pltpu.async_copy(aadj_hbm.at[eidx.at[sl]], arow.at[sl], asem))

    pltpu.sync_copy(sen_hbm.at[pl.ds(base, BW)], senv)
    pltpu.sync_copy(lloc_hbm.at[pl.ds(base * 3, EW)], llocv)
    pltpu.sync_copy(aloc_hbm.at[pl.ds(base * 3, EW)], alocv)
    pltpu.sync_copy(ttc_hbm, ttcv)

    def ttc_group(g, carry):
        sidx = senv[pl.ds(g * L, L)]
        outt[pl.ds(g * L, L)] = plsc.load_gather(ttcv, [sidx])
        return carry

    lax.fori_loop(0, BW // L, ttc_group, 0)

    for cp in copies:
        cp.wait()

    def combine(g, carry):
        sl = pl.ds(g * L, L)
        m = g * L + lane
        pos = m + 3 * (m // 3)
        plsc.store_scatter(outv, [pos], lrow[sl] + llocv[sl])
        plsc.store_scatter(outv, [pos + 3], arow[sl] + alocv[sl])
        return carry

    lax.fori_loop(0, EW // L, combine, 0)

    pltpu.sync_copy(outv, out_hbm.at[pl.ds(base * 6, BW * 6)])
    pltpu.sync_copy(outt, tout_hbm.at[pl.ds(base, BW)])


@jax.jit
def kernel(linear_velocities_local, angular_velocities_local,
           linear_velocity_adjustment, angular_velocity_adjustment,
           time_to_center_pixel_adjustment, cam_idx, sensor_idx):
    ttc_pad = jnp.pad(time_to_center_pixel_adjustment, (0, 32 - 26))
    run = pl.kernel(
        _body,
        out_type=(
            jax.ShapeDtypeStruct((B * 6,), jnp.float32),
            jax.ShapeDtypeStruct((B,), jnp.float32),
        ),
        mesh=plsc.VectorSubcoreMesh(core_axis_name="c", subcore_axis_name="s"),
        compiler_params=pltpu.CompilerParams(needs_layout_passes=False),
        scratch_types=[
            pltpu.VMEM((BW,), jnp.int32),
            pltpu.VMEM((BW,), jnp.int32),
            pltpu.VMEM((EW,), jnp.int32),
            pltpu.VMEM((EW,), jnp.float32),
            pltpu.VMEM((EW,), jnp.float32),
            pltpu.VMEM((EW,), jnp.float32),
            pltpu.VMEM((EW,), jnp.float32),
            pltpu.VMEM((BW * 6,), jnp.float32),
            pltpu.VMEM((BW,), jnp.float32),
            pltpu.VMEM((32,), jnp.float32),
            pltpu.SemaphoreType.DMA,
            pltpu.SemaphoreType.DMA,
        ],
    )
    out_flat, ttc = run(
        linear_velocities_local.reshape(-1),
        angular_velocities_local.reshape(-1),
        linear_velocity_adjustment.reshape(-1),
        angular_velocity_adjustment.reshape(-1),
        ttc_pad, cam_idx.astype(jnp.int32), sensor_idx.astype(jnp.int32),
    )
    return out_flat.reshape(B, 6), ttc


# speedup vs baseline: 48.4935x; 48.4935x over previous
import jax
import jax.numpy as jnp
from jax import lax
from jax.experimental import pallas as pl
from jax.experimental.pallas import tpu as pltpu
from jax.experimental.pallas import tpu_sc as plsc

L = 16
NW = 32
B = 16384
BW = B // NW
ICH = 128
NCH = BW // ICH


def _body(lc0, lc1, lc2, ac0, ac1, ac2,
          ll0, ll1, ll2, al0, al1, al2,
          ttc_hbm, cam_hbm, sen_hbm,
          out_hbm, tout_hbm,
          camv, senv, gat, loc, outv, outt, ttcv, sem0, sem1):
    wid = lax.axis_index("s") * 2 + lax.axis_index("c")
    base = wid * BW
    cols = (lc0, lc1, lc2, ac0, ac1, ac2)
    locs = (ll0, ll1, ll2, al0, al1, al2)
    sems = (sem0, sem1)

    pltpu.sync_copy(cam_hbm.at[pl.ds(wid * NCH, NCH)], camv)
    copies = []
    for j in range(NCH):
        idx = camv.at[j]
        dst = pl.ds(j * ICH, ICH)
        for c in range(6):
            copies.append(
                pltpu.async_copy(cols[c].at[idx], gat.at[c].at[dst], sems[c % 2]))

    pltpu.sync_copy(sen_hbm.at[pl.ds(base, BW)], senv)
    for c in range(6):
        pltpu.sync_copy(locs[c].at[pl.ds(base, BW)], loc.at[c])
    pltpu.sync_copy(ttc_hbm, ttcv)

    def ttc_group(g, carry):
        sidx = senv[pl.ds(g * L, L)]
        outt[pl.ds(g * L, L)] = plsc.load_gather(ttcv, [sidx])
        return carry

    lax.fori_loop(0, BW // L, ttc_group, 0)

    for cp in copies:
        cp.wait()

    def combine(g, carry):
        sl = pl.ds(g * L, L)
        for c in range(6):
            outv[c, sl] = gat[c, sl] + loc[c, sl]
        return carry

    lax.fori_loop(0, BW // L, combine, 0)

    for c in range(6):
        pltpu.sync_copy(outv.at[c], out_hbm.at[c].at[pl.ds(base, BW)])
    pltpu.sync_copy(outt, tout_hbm.at[pl.ds(base, BW)])


@jax.jit
def kernel(linear_velocities_local, angular_velocities_local,
           linear_velocity_adjustment, angular_velocity_adjustment,
           time_to_center_pixel_adjustment, cam_idx, sensor_idx):
    run = pl.kernel(
        _body,
        out_type=(
            jax.ShapeDtypeStruct((6, B), jnp.float32),
            jax.ShapeDtypeStruct((B,), jnp.float32),
        ),
        mesh=plsc.VectorSubcoreMesh(core_axis_name="c", subcore_axis_name="s"),
        compiler_params=pltpu.CompilerParams(
            needs_layout_passes=False, use_tc_tiling_on_sc=False),
        scratch_types=[
            pltpu.VMEM((NCH, ICH), jnp.int32),
            pltpu.VMEM((BW,), jnp.int32),
            pltpu.VMEM((6, BW), jnp.float32),
            pltpu.VMEM((6, BW), jnp.float32),
            pltpu.VMEM((6, BW), jnp.float32),
            pltpu.VMEM((BW,), jnp.float32),
            pltpu.VMEM((26,), jnp.float32),
            pltpu.SemaphoreType.DMA,
            pltpu.SemaphoreType.DMA,
        ],
    )
    out_t, ttc = run(
        linear_velocity_adjustment[:, 0], linear_velocity_adjustment[:, 1],
        linear_velocity_adjustment[:, 2],
        angular_velocity_adjustment[:, 0], angular_velocity_adjustment[:, 1],
        angular_velocity_adjustment[:, 2],
        linear_velocities_local[:, 0], linear_velocities_local[:, 1],
        linear_velocities_local[:, 2],
        angular_velocities_local[:, 0], angular_velocities_local[:, 1],
        angular_velocities_local[:, 2],
        time_to_center_pixel_adjustment,
        cam_idx.astype(jnp.int32).reshape(B // ICH, ICH),
        sensor_idx.astype(jnp.int32),
    )
    return out_t.T, ttc


# speedup vs baseline: 62.1092x; 1.2808x over previous
import jax
import jax.numpy as jnp
from jax import lax
from jax.experimental import pallas as pl
from jax.experimental.pallas import tpu as pltpu
from jax.experimental.pallas import tpu_sc as plsc

L = 16
NW = 32
B = 16384
N = 1000000
BW = B // NW
ICH = 128
NCH = BW // ICH


def _body(ladj_hbm, aadj_hbm, lloc_hbm, aloc_hbm, ttc_hbm, cam_hbm, sen_hbm,
          out_hbm, tout_hbm,
          camv, eidx1, eidx2, senv, gat, loc, outv, outt, ttcv, sem0, sem1):
    wid = lax.axis_index("s") * 2 + lax.axis_index("c")
    base = wid * BW

    pltpu.sync_copy(cam_hbm.at[pl.ds(wid * NCH, NCH)], camv)

    def shift(g, carry):
        j = g // (ICH // L)
        sl = pl.ds((g % (ICH // L)) * L, L)
        v = camv[j, sl]
        eidx1[j, sl] = v + N
        eidx2[j, sl] = v + 2 * N
        return carry

    lax.fori_loop(0, NCH * (ICH // L), shift, 0)

    idxs = (camv, eidx1, eidx2)
    sems = (sem0, sem1)
    copies = []
    for j in range(NCH):
        dst = pl.ds(j * ICH, ICH)
        for c in range(3):
            copies.append(pltpu.async_copy(
                ladj_hbm.at[idxs[c].at[j]], gat.at[c].at[dst], sems[0]))
            copies.append(pltpu.async_copy(
                aadj_hbm.at[idxs[c].at[j]], gat.at[c + 3].at[dst], sems[1]))

    pltpu.sync_copy(sen_hbm.at[pl.ds(base, BW)], senv)
    for c in range(3):
        pltpu.sync_copy(lloc_hbm.at[pl.ds(c * B + base, BW)], loc.at[c])
        pltpu.sync_copy(aloc_hbm.at[pl.ds(c * B + base, BW)], loc.at[c + 3])
    pltpu.sync_copy(ttc_hbm, ttcv)

    def ttc_group(g, carry):
        sidx = senv[pl.ds(g * L, L)]
        outt[pl.ds(g * L, L)] = plsc.load_gather(ttcv, [sidx])
        return carry

    lax.fori_loop(0, BW // L, ttc_group, 0)

    for cp in copies:
        cp.wait()

    def combine(g, carry):
        sl = pl.ds(g * L, L)
        for c in range(6):
            outv[c, sl] = gat[c, sl] + loc[c, sl]
        return carry

    lax.fori_loop(0, BW // L, combine, 0)

    for c in range(6):
        pltpu.sync_copy(outv.at[c], out_hbm.at[c].at[pl.ds(base, BW)])
    pltpu.sync_copy(outt, tout_hbm.at[pl.ds(base, BW)])


@jax.jit
def kernel(linear_velocities_local, angular_velocities_local,
           linear_velocity_adjustment, angular_velocity_adjustment,
           time_to_center_pixel_adjustment, cam_idx, sensor_idx):
    run = pl.kernel(
        _body,
        out_type=(
            jax.ShapeDtypeStruct((6, B), jnp.float32),
            jax.ShapeDtypeStruct((B,), jnp.float32),
        ),
        mesh=plsc.VectorSubcoreMesh(core_axis_name="c", subcore_axis_name="s"),
        compiler_params=pltpu.CompilerParams(
            needs_layout_passes=False, use_tc_tiling_on_sc=False),
        scratch_types=[
            pltpu.VMEM((NCH, ICH), jnp.int32),
            pltpu.VMEM((NCH, ICH), jnp.int32),
            pltpu.VMEM((NCH, ICH), jnp.int32),
            pltpu.VMEM((BW,), jnp.int32),
            pltpu.VMEM((6, BW), jnp.float32),
            pltpu.VMEM((6, BW), jnp.float32),
            pltpu.VMEM((6, BW), jnp.float32),
            pltpu.VMEM((BW,), jnp.float32),
            pltpu.VMEM((26,), jnp.float32),
            pltpu.SemaphoreType.DMA,
            pltpu.SemaphoreType.DMA,
        ],
    )
    out_t, ttc = run(
        linear_velocity_adjustment.T.reshape(-1),
        angular_velocity_adjustment.T.reshape(-1),
        linear_velocities_local.T.reshape(-1),
        angular_velocities_local.T.reshape(-1),
        time_to_center_pixel_adjustment,
        cam_idx.astype(jnp.int32).reshape(B // ICH, ICH),
        sensor_idx.astype(jnp.int32),
    )
    return out_t.T, ttc


# speedup vs baseline: 64.3551x; 1.0362x over previous
import jax
import jax.numpy as jnp
from jax import lax
from jax.experimental import pallas as pl
from jax.experimental.pallas import tpu as pltpu
from jax.experimental.pallas import tpu_sc as plsc
from jax.experimental.layout import Layout, with_layout_constraint

L = 16
NW = 32
B = 16384
N = 1000000
NBLK = 7813
NFR = NBLK * 64
BW = B // NW
ICH = 128
NCH = BW // ICH


def _body(ladj_hbm, aadj_hbm, lloc_hbm, aloc_hbm, ttc_hbm, cam_hbm, sen_hbm,
          out_hbm, tout_hbm,
          camv, f0, f1, f2, subv, senv, gat, loc, outv, outt, ttcv,
          sem0, sem1):
    wid = lax.axis_index("s") * 2 + lax.axis_index("c")
    base = wid * BW

    pltpu.sync_copy(cam_hbm.at[pl.ds(wid * NCH, NCH)], camv)
    lane = lax.iota(jnp.int32, L)

    def build(g, carry):
        j = g // (ICH // L)
        sl = pl.ds((g % (ICH // L)) * L, L)
        cam = camv[j, sl]
        f = 64 * (cam >> 7) + ((cam & 127) >> 3)
        f0[j, sl] = f
        f1[j, sl] = f + 16
        f2[j, sl] = f + 32
        subv[pl.ds(g * L, L)] = cam & 7
        return carry

    lax.fori_loop(0, NCH * (ICH // L), build, 0)

    idxs = (f0, f1, f2)
    copies = []
    for j in range(NCH):
        dst = pl.ds(j * ICH, ICH)
        for c in range(3):
            copies.append(pltpu.async_copy(
                ladj_hbm.at[idxs[c].at[j]], gat.at[c].at[dst], sem0))
            copies.append(pltpu.async_copy(
                aadj_hbm.at[idxs[c].at[j]], gat.at[c + 3].at[dst], sem1))

    pltpu.sync_copy(sen_hbm.at[pl.ds(base, BW)], senv)
    for c in range(3):
        pltpu.sync_copy(lloc_hbm.at[pl.ds(c * B + base, BW)], loc.at[c])
        pltpu.sync_copy(aloc_hbm.at[pl.ds(c * B + base, BW)], loc.at[c + 3])
    pltpu.sync_copy(ttc_hbm, ttcv)

    def ttc_group(g, carry):
        sidx = senv[pl.ds(g * L, L)]
        outt[pl.ds(g * L, L)] = plsc.load_gather(ttcv, [sidx])
        return carry

    lax.fori_loop(0, BW // L, ttc_group, 0)

    for cp in copies:
        cp.wait()

    def combine(g, carry):
        sl = pl.ds(g * L, L)
        rows = g * L + lane
        sub = subv[sl]
        for c in range(6):
            v = plsc.load_gather(gat.at[c], [rows, sub])
            outv[c, sl] = v + loc[c, sl]
        return carry

    lax.fori_loop(0, BW // L, combine, 0)

    for c in range(6):
        pltpu.sync_copy(outv.at[c], out_hbm.at[c].at[pl.ds(base, BW)])
    pltpu.sync_copy(outt, tout_hbm.at[pl.ds(base, BW)])


def _fake_row_view(table):
    y = jnp.pad(table, ((0, NBLK * 128 - N), (0, 1)))
    y = y.reshape(NBLK, 128, 4)
    y = with_layout_constraint(y, Layout(major_to_minor=(0, 2, 1)))
    return y.transpose(0, 2, 1).reshape(NFR, 8)


@jax.jit
def kernel(linear_velocities_local, angular_velocities_local,
           linear_velocity_adjustment, angular_velocity_adjustment,
           time_to_center_pixel_adjustment, cam_idx, sensor_idx):
    run = pl.kernel(
        _body,
        out_type=(
            jax.ShapeDtypeStruct((6, B), jnp.float32),
            jax.ShapeDtypeStruct((B,), jnp.float32),
        ),
        mesh=plsc.VectorSubcoreMesh(core_axis_name="c", subcore_axis_name="s"),
        compiler_params=pltpu.CompilerParams(
            needs_layout_passes=False, use_tc_tiling_on_sc=False),
        scratch_types=[
            pltpu.VMEM((NCH, ICH), jnp.int32),
            pltpu.VMEM((NCH, ICH), jnp.int32),
            pltpu.VMEM((NCH, ICH), jnp.int32),
            pltpu.VMEM((NCH, ICH), jnp.int32),
            pltpu.VMEM((BW,), jnp.int32),
            pltpu.VMEM((BW,), jnp.int32),
            pltpu.VMEM((6, BW, 8), jnp.float32),
            pltpu.VMEM((6, BW), jnp.float32),
            pltpu.VMEM((6, BW), jnp.float32),
            pltpu.VMEM((BW,), jnp.float32),
            pltpu.VMEM((26,), jnp.float32),
            pltpu.SemaphoreType.DMA,
            pltpu.SemaphoreType.DMA,
        ],
    )
    out_t, ttc = run(
        _fake_row_view(linear_velocity_adjustment),
        _fake_row_view(angular_velocity_adjustment),
        linear_velocities_local.T.reshape(-1),
        angular_velocities_local.T.reshape(-1),
        time_to_center_pixel_adjustment,
        cam_idx.astype(jnp.int32).reshape(B // ICH, ICH),
        sensor_idx.astype(jnp.int32),
    )
    return out_t.T, ttc


# speedup vs baseline: 112.2953x; 1.7449x over previous
import jax
import jax.numpy as jnp
from jax import lax
from jax.experimental import pallas as pl
from jax.experimental.pallas import tpu as pltpu
from jax.experimental.pallas import tpu_sc as plsc
from jax.experimental.layout import Layout, with_layout_constraint

L = 16
NW = 32
B = 16384
N = 1000000
NBLK = 7816
NFW = NBLK * 512
BW = B // NW
ICH = 128
NCH = BW // ICH


def _body(ladj_hbm, aadj_hbm, lloc_hbm, aloc_hbm, ttc_hbm, cam_hbm, sen_hbm,
          out_hbm, tout_hbm,
          camv, w0, w1, w2, senv, gat, loc, outv, outt, ttcv, sem0, sem1):
    wid = lax.axis_index("s") * 2 + lax.axis_index("c")
    base = wid * BW

    pltpu.sync_copy(cam_hbm.at[pl.ds(wid * NCH, NCH)], camv)
    lane = lax.iota(jnp.int32, L)

    def build(g, carry):
        j = g // (ICH // L)
        sl = pl.ds((g % (ICH // L)) * L, L)
        cam = camv[j, sl]
        w = 512 * (cam >> 7) + (cam & 127)
        w0[j, sl] = w
        w1[j, sl] = w + 128
        w2[j, sl] = w + 256
        return carry

    lax.fori_loop(0, NCH * (ICH // L), build, 0)

    idxs = (w0, w1, w2)
    copies = []
    for j in range(NCH):
        dst = pl.ds(j * ICH, ICH)
        for c in range(3):
            copies.append(pltpu.async_copy(
                ladj_hbm.at[idxs[c].at[j]], gat.at[c].at[dst], sem0))
            copies.append(pltpu.async_copy(
                aadj_hbm.at[idxs[c].at[j]], gat.at[c + 3].at[dst], sem1))

    pltpu.sync_copy(sen_hbm.at[pl.ds(base, BW)], senv)
    for c in range(3):
        pltpu.sync_copy(lloc_hbm.at[pl.ds(c * B + base, BW)], loc.at[c])
        pltpu.sync_copy(aloc_hbm.at[pl.ds(c * B + base, BW)], loc.at[c + 3])
    pltpu.sync_copy(ttc_hbm, ttcv)

    def ttc_group(g, carry):
        sidx = senv[pl.ds(g * L, L)]
        outt[pl.ds(g * L, L)] = plsc.load_gather(ttcv, [sidx])
        return carry

    lax.fori_loop(0, BW // L, ttc_group, 0)

    for cp in copies:
        cp.wait()

    def combine(g, carry):
        sl = pl.ds(g * L, L)
        for c in range(6):
            outv[c, sl] = gat[c, sl] + loc[c, sl]
        return carry

    lax.fori_loop(0, BW // L, combine, 0)

    for c in range(6):
        pltpu.sync_copy(outv.at[c], out_hbm.at[c].at[pl.ds(base, BW)])
    pltpu.sync_copy(outt, tout_hbm.at[pl.ds(base, BW)])


def _flat_view(table):
    y = jnp.pad(table, ((0, NBLK * 128 - N), (0, 1)))
    y = y.reshape(NBLK, 128, 4)
    y = with_layout_constraint(y, Layout(major_to_minor=(0, 2, 1)))
    return y.transpose(0, 2, 1).reshape(-1)


@jax.jit
def kernel(linear_velocities_local, angular_velocities_local,
           linear_velocity_adjustment, angular_velocity_adjustment,
           time_to_center_pixel_adjustment, cam_idx, sensor_idx):
    run = pl.kernel(
        _body,
        out_type=(
            jax.ShapeDtypeStruct((6, B), jnp.float32),
            jax.ShapeDtypeStruct((B,), jnp.float32),
        ),
        mesh=plsc.VectorSubcoreMesh(core_axis_name="c", subcore_axis_name="s"),
        compiler_params=pltpu.CompilerParams(
            needs_layout_passes=False, use_tc_tiling_on_sc=False),
        scratch_types=[
            pltpu.VMEM((NCH, ICH), jnp.int32),
            pltpu.VMEM((NCH, ICH), jnp.int32),
            pltpu.VMEM((NCH, ICH), jnp.int32),
            pltpu.VMEM((NCH, ICH), jnp.int32),
            pltpu.VMEM((BW,), jnp.int32),
            pltpu.VMEM((6, BW), jnp.float32),
            pltpu.VMEM((6, BW), jnp.float32),
            pltpu.VMEM((6, BW), jnp.float32),
            pltpu.VMEM((BW,), jnp.float32),
            pltpu.VMEM((26,), jnp.float32),
            pltpu.SemaphoreType.DMA,
            pltpu.SemaphoreType.DMA,
        ],
    )
    out_t, ttc = run(
        _flat_view(linear_velocity_adjustment),
        _flat_view(angular_velocity_adjustment),
        linear_velocities_local.T.reshape(-1),
        angular_velocities_local.T.reshape(-1),
        time_to_center_pixel_adjustment,
        cam_idx.astype(jnp.int32).reshape(B // ICH, ICH),
        sensor_idx.astype(jnp.int32),
    )
    return out_t.T, ttc


# speedup vs baseline: 112.5850x; 1.0026x over previous
import functools

import jax
import jax.numpy as jnp
from jax import lax
from jax.experimental import pallas as pl
from jax.experimental.pallas import tpu as pltpu
from jax.experimental.pallas import tpu_sc as plsc
from jax.experimental.layout import Layout, with_layout_constraint

L = 16
NW = 32
B = 16384
N = 1000000
NBLK = 7816
NFW = NBLK * 512
BW = B // NW
ICH = 128
NCH = BW // ICH


def _stage_and_gather(adj_hbm, loc_hbm, cam_hbm, wid, camv, w0, w1, w2, gat,
                      loc, sem):
    base = wid * BW
    pltpu.sync_copy(cam_hbm.at[pl.ds(wid * NCH, NCH)], camv)
    lane = lax.iota(jnp.int32, L)

    def build(g, carry):
        j = g // (ICH // L)
        sl = pl.ds((g % (ICH // L)) * L, L)
        cam = camv[j, sl]
        w = 512 * (cam >> 7) + (cam & 127)
        w0[j, sl] = w
        w1[j, sl] = w + 128
        w2[j, sl] = w + 256
        return carry

    lax.fori_loop(0, NCH * (ICH // L), build, 0)

    idxs = (w0, w1, w2)
    copies = []
    for j in range(NCH):
        dst = pl.ds(j * ICH, ICH)
        for c in range(3):
            copies.append(pltpu.async_copy(
                adj_hbm.at[idxs[c].at[j]], gat.at[c].at[dst], sem))
    for c in range(3):
        pltpu.sync_copy(loc_hbm.at[pl.ds(c * B + base, BW)], loc.at[c])
    return copies


def _combine_and_store(wid, gat, loc, outv, out_hbm):
    base = wid * BW

    def combine(g, carry):
        sl = pl.ds(g * L, L)
        for c in range(3):
            outv[c, sl] = gat[c, sl] + loc[c, sl]
        return carry

    lax.fori_loop(0, BW // L, combine, 0)
    for c in range(3):
        pltpu.sync_copy(outv.at[c], out_hbm.at[c].at[pl.ds(base, BW)])


def _body_lin(ladj_hbm, lloc_hbm, cam_hbm, out_hbm,
              camv, w0, w1, w2, gat, loc, outv, sem):
    wid = lax.axis_index("s") * 2 + lax.axis_index("c")
    copies = _stage_and_gather(ladj_hbm, lloc_hbm, cam_hbm, wid,
                               camv, w0, w1, w2, gat, loc, sem)
    for cp in copies:
        cp.wait()
    _combine_and_store(wid, gat, loc, outv, out_hbm)


def _body_ang(aadj_hbm, aloc_hbm, cam_hbm, ttc_hbm, sen_hbm,
              out_hbm, tout_hbm,
              camv, w0, w1, w2, senv, gat, loc, outv, outt, ttcv, sem):
    wid = lax.axis_index("s") * 2 + lax.axis_index("c")
    base = wid * BW
    copies = _stage_and_gather(aadj_hbm, aloc_hbm, cam_hbm, wid,
                               camv, w0, w1, w2, gat, loc, sem)
    pltpu.sync_copy(sen_hbm.at[pl.ds(base, BW)], senv)
    pltpu.sync_copy(ttc_hbm, ttcv)

    def ttc_group(g, carry):
        sidx = senv[pl.ds(g * L, L)]
        outt[pl.ds(g * L, L)] = plsc.load_gather(ttcv, [sidx])
        return carry

    lax.fori_loop(0, BW // L, ttc_group, 0)
    for cp in copies:
        cp.wait()
    _combine_and_store(wid, gat, loc, outv, out_hbm)
    pltpu.sync_copy(outt, tout_hbm.at[pl.ds(base, BW)])


def _flat_view(table):
    y = jnp.pad(table, ((0, NBLK * 128 - N), (0, 1)))
    y = y.reshape(NBLK, 128, 4)
    y = with_layout_constraint(y, Layout(major_to_minor=(0, 2, 1)))
    return y.transpose(0, 2, 1).reshape(-1)


_MESH = dict(
    mesh=plsc.VectorSubcoreMesh(core_axis_name="c", subcore_axis_name="s"),
    compiler_params=pltpu.CompilerParams(
        needs_layout_passes=False, use_tc_tiling_on_sc=False),
)
_COMMON_SCRATCH = [
    pltpu.VMEM((NCH, ICH), jnp.int32),
    pltpu.VMEM((NCH, ICH), jnp.int32),
    pltpu.VMEM((NCH, ICH), jnp.int32),
    pltpu.VMEM((NCH, ICH), jnp.int32),
]


@jax.jit
def kernel(linear_velocities_local, angular_velocities_local,
           linear_velocity_adjustment, angular_velocity_adjustment,
           time_to_center_pixel_adjustment, cam_idx, sensor_idx):
    run_lin = pl.kernel(
        _body_lin,
        out_type=jax.ShapeDtypeStruct((3, B), jnp.float32),
        scratch_types=_COMMON_SCRATCH + [
            pltpu.VMEM((3, BW), jnp.float32),
            pltpu.VMEM((3, BW), jnp.float32),
            pltpu.VMEM((3, BW), jnp.float32),
            pltpu.SemaphoreType.DMA,
        ],
        **_MESH,
    )
    run_ang = pl.kernel(
        _body_ang,
        out_type=(
            jax.ShapeDtypeStruct((3, B), jnp.float32),
            jax.ShapeDtypeStruct((B,), jnp.float32),
        ),
        scratch_types=_COMMON_SCRATCH + [
            pltpu.VMEM((BW,), jnp.int32),
            pltpu.VMEM((3, BW), jnp.float32),
            pltpu.VMEM((3, BW), jnp.float32),
            pltpu.VMEM((3, BW), jnp.float32),
            pltpu.VMEM((BW,), jnp.float32),
            pltpu.VMEM((26,), jnp.float32),
            pltpu.SemaphoreType.DMA,
        ],
        **_MESH,
    )
    cam2d = cam_idx.astype(jnp.int32).reshape(B // ICH, ICH)
    out_lin = run_lin(
        _flat_view(linear_velocity_adjustment),
        linear_velocities_local.T.reshape(-1),
        cam2d,
    )
    out_ang, ttc = run_ang(
        _flat_view(angular_velocity_adjustment),
        angular_velocities_local.T.reshape(-1),
        cam2d,
        time_to_center_pixel_adjustment,
        sensor_idx.astype(jnp.int32),
    )
    return jnp.concatenate([out_lin, out_ang], axis=0).T, ttc
